# baseline (speedup 1.0000x reference)
import functools

import jax
import jax.numpy as jnp
from jax import lax
from jax.experimental import pallas as pl
from jax.experimental.pallas import tpu as pltpu
from jax.experimental.pallas import tpu_sc as plsc

N, D_IN, D_OUT, E = 4096, 1024, 1024, 16
L = 16
NC, NS = 2, 16
BN = 512
LOG_BN = 9
NB = N // BN + E
P = NB * BN
TPW1 = N // NS
TPW3 = N // (NC * NS)
RPD = 16


NBUF = 4


def _route_body(idx_hbm, x_hbm, x_pad_hbm, dest_hbm, be_hbm,
                idx_v, rank_v, cnt_v, allcnt_v, tb_v, dest_v, bev,
                xbuf0, xbuf1, xbuf2, xbuf3, shared_cnt,
                lsem0, lsem1, lsem2, lsem3, ssem0, ssem1, ssem2, ssem3,
                dsem):
    cid = lax.axis_index("c")
    sid = lax.axis_index("s")
    base = sid * TPW1
    half = cid * (TPW1 // 2)
    lane = lax.iota(jnp.int32, L)

    nch = TPW1 // 2 // RPD
    bufs = (xbuf0, xbuf1, xbuf2, xbuf3)
    lsems = (lsem0, lsem1, lsem2, lsem3)
    ssems = (ssem0, ssem1, ssem2, ssem3)
    ld = [None] * NBUF
    st = [None] * NBUF

    def start_load(j):
        b = j % NBUF
        ld[b] = pltpu.async_copy(
            x_hbm.at[pl.ds(base + half + j * RPD, RPD)], bufs[b], lsems[b])

    for j in range(NBUF):
        start_load(j)

    pltpu.sync_copy(idx_hbm.at[pl.ds(base, TPW1)], idx_v)

    def vreg_body(j, cnt):
        ev = idx_v[pl.ds(j * L, L)]
        rank = jnp.zeros((L,), jnp.int32)
        for k in range(L):
            ek = jnp.sum(jnp.where(lane == k, ev, 0))
            is_ek = (lane == ek)
            rk = jnp.sum(jnp.where(is_ek, cnt, 0))
            rank = jnp.where(lane == k, rk, rank)
            cnt = cnt + is_ek.astype(jnp.int32)
        rank_v[pl.ds(j * L, L)] = rank
        return cnt

    cnt = lax.fori_loop(0, TPW1 // L, vreg_body, jnp.zeros((L,), jnp.int32))

    cnt_v[...] = cnt
    pltpu.sync_copy(cnt_v, shared_cnt.at[pl.ds(sid * L, L)])
    plsc.subcore_barrier()
    pltpu.sync_copy(shared_cnt, allcnt_v)

    def acc_body(t, carry):
        tot, pri = carry
        row = allcnt_v[pl.ds(t * L, L)]
        tot = tot + row
        pri = pri + jnp.where(t < sid, row, 0)
        return tot, pri

    zeros = jnp.zeros((L,), jnp.int32)
    tot, pri = lax.fori_loop(0, NS, acc_body, (zeros, zeros))

    padded = ((tot + (BN - 1)) >> LOG_BN) << LOG_BN
    pend = plsc.cumsum(padded)
    pstart = pend - padded
    tb_v[...] = pstart + pri

    for j in range(TPW1 // L):
        ev = idx_v[pl.ds(j * L, L)]
        rk = rank_v[pl.ds(j * L, L)]
        dest_v[pl.ds(j * L, L)] = plsc.load_gather(tb_v, [ev]) + rk

    dest_wr = pltpu.async_copy(dest_v.at[pl.ds(half, TPW1 // 2)],
                               dest_hbm.at[pl.ds(base + half, TPW1 // 2)],
                               dsem)

    st_desc = [None] * nch
    waited = [False] * nch
    for j in range(nch):
        b = j % NBUF
        ld[b].wait()
        dv = dest_v[pl.ds(half + j * RPD, RPD)]
        st_desc[j] = pltpu.async_copy(bufs[b], x_pad_hbm.at[dv], ssems[b])
        nxt = j + NBUF - 1
        if NBUF <= nxt < nch:
            st_desc[j - 1].wait()
            waited[j - 1] = True
            start_load(nxt)
    for j in range(nch):
        if st_desc[j] is not None and not waited[j]:
            st_desc[j].wait()
    dest_wr.wait()

    @pl.when((sid == 0) & (cid == 0))
    def _():
        pend_blk = pend >> LOG_BN
        for v in range(NB // L):
            blk = lane + v * L
            acc = jnp.zeros((L,), jnp.int32)
            for e in range(E):
                pe = jnp.sum(jnp.where(lane == e, pend_blk, 0))
                acc = acc + (pe <= blk).astype(jnp.int32)
            bev[pl.ds(v * L, L)] = jnp.minimum(acc, E - 1)
        nblk = jnp.sum(jnp.where(lane == L - 1, pend_blk, 0))
        bev[pl.ds(NB, L)] = jnp.where(lane == 0, nblk, 0)
        pltpu.sync_copy(bev, be_hbm)


def _gmm_body(be_ref, x_ref, w_ref, o_ref):
    @pl.when(pl.program_id(0) < be_ref[NB])
    def _():
        o_ref[...] = jnp.dot(x_ref[...], w_ref[0],
                             preferred_element_type=jnp.float32)


NSPLIT = 2
DS = D_OUT // NSPLIT


def _gather_body(dest_hbm, out_pad_hbm, out_hbm, dest_v, buf0, buf1, buf2,
                 buf3, gsem0, gsem1, gsem2, gsem3, wsem0, wsem1, wsem2,
                 wsem3):
    wid = lax.axis_index("s") * NC + lax.axis_index("c")
    base = wid * TPW3
    pltpu.sync_copy(dest_hbm.at[pl.ds(base, TPW3)], dest_v)

    nch = TPW3 // RPD
    bufs = (buf0, buf1, buf2, buf3)
    gsems = (gsem0, gsem1, gsem2, gsem3)
    wsems = (wsem0, wsem1, wsem2, wsem3)
    g = [None] * NBUF

    def start_gather(j):
        b = j % NBUF
        dv = dest_v[pl.ds(j * RPD, RPD)]
        g[b] = pltpu.async_copy(out_pad_hbm.at[dv], bufs[b], gsems[b])

    for j in range(min(NBUF, nch)):
        start_gather(j)

    w_desc = [None] * nch
    waited = [False] * nch
    for j in range(nch):
        b = j % NBUF
        g[b].wait()
        w_desc[j] = pltpu.async_copy(
            bufs[b], out_hbm.at[pl.ds(base + j * RPD, RPD)], wsems[b])
        nxt = j + NBUF - 1
        if NBUF <= nxt < nch:
            w_desc[j - 1].wait()
            waited[j - 1] = True
            start_gather(nxt)
    for j in range(nch):
        if w_desc[j] is not None and not waited[j]:
            w_desc[j].wait()


@jax.jit
def kernel(x, indices, weight):
    route = pl.kernel(
        _route_body,
        out_type=(
            jax.ShapeDtypeStruct((P, D_IN), jnp.float32),
            jax.ShapeDtypeStruct((N,), jnp.int32),
            jax.ShapeDtypeStruct((NB + L,), jnp.int32),
        ),
        mesh=plsc.VectorSubcoreMesh(
            core_axis_name="c", subcore_axis_name="s", num_cores=NC),
        compiler_params=pltpu.CompilerParams(needs_layout_passes=False),
        scratch_types=(
            pltpu.VMEM((TPW1,), jnp.int32),
            pltpu.VMEM((TPW1,), jnp.int32),
            pltpu.VMEM((L,), jnp.int32),
            pltpu.VMEM((NS * L,), jnp.int32),
            pltpu.VMEM((L,), jnp.int32),
            pltpu.VMEM((TPW1,), jnp.int32),
            pltpu.VMEM((NB + L,), jnp.int32),
            pltpu.VMEM((RPD, D_IN), jnp.float32),
            pltpu.VMEM((RPD, D_IN), jnp.float32),
            pltpu.VMEM((RPD, D_IN), jnp.float32),
            pltpu.VMEM((RPD, D_IN), jnp.float32),
            pltpu.VMEM_SHARED((NS * L,), jnp.int32),
        ) + (pltpu.SemaphoreType.DMA,) * (2 * NBUF + 1),
    )
    x_pad, dest, block_expert = route(indices, x)

    out_pad = pl.pallas_call(
        _gmm_body,
        grid_spec=pltpu.PrefetchScalarGridSpec(
            num_scalar_prefetch=1,
            grid=(NB, NSPLIT),
            in_specs=[
                pl.BlockSpec((BN, D_IN),
                             lambda i, j, be: (jnp.minimum(i, be[NB] - 1),
                                               0)),
                pl.BlockSpec((1, D_IN, DS),
                             lambda i, j, be:
                             (be[jnp.minimum(i, be[NB] - 1)], 0, j)),
            ],
            out_specs=pl.BlockSpec((BN, DS),
                                   lambda i, j, be:
                                   (jnp.minimum(i, be[NB] - 1), j)),
        ),
        out_shape=jax.ShapeDtypeStruct((P, D_OUT), jnp.float32),
    )(block_expert, x_pad, weight)

    gather_back = pl.kernel(
        _gather_body,
        out_type=jax.ShapeDtypeStruct((N, D_OUT), jnp.float32),
        mesh=plsc.VectorSubcoreMesh(
            core_axis_name="c", subcore_axis_name="s", num_cores=NC),
        compiler_params=pltpu.CompilerParams(needs_layout_passes=False),
        scratch_types=(
            pltpu.VMEM((TPW3,), jnp.int32),
        ) + (pltpu.VMEM((RPD, D_OUT), jnp.float32),) * NBUF
          + (pltpu.SemaphoreType.DMA,) * (2 * NBUF),
    )
    return gather_back(dest, out_pad)


# speedup vs baseline: 1.3992x; 1.3992x over previous
import functools

import jax
import jax.numpy as jnp
from jax import lax
from jax.experimental import pallas as pl
from jax.experimental.pallas import tpu as pltpu
from jax.experimental.pallas import tpu_sc as plsc

N, D_IN, D_OUT, E = 4096, 1024, 1024, 16
L = 16
NC, NS = 2, 16
BN = 512
LOG_BN = 9
NB = N // BN + E
P = NB * BN
TPW1 = N // NS
TPW3 = N // (NC * NS)
RPD = 16


NBUF = 7


def _route_body(idx_hbm, x_hbm, x_pad_hbm, dest_hbm, be_hbm,
                idx_v, rank_v, cnt_v, allcnt_v, tb_v, dest_v, bev,
                *ring):
    cid = lax.axis_index("c")
    sid = lax.axis_index("s")
    base = sid * TPW1
    half = cid * (TPW1 // 2)
    lane = lax.iota(jnp.int32, L)

    nch = TPW1 // 2 // RPD
    bufs = ring[:NBUF]
    shared_cnt = ring[NBUF]
    lsems = ring[NBUF + 1:2 * NBUF + 1]
    ssems = ring[2 * NBUF + 1:3 * NBUF + 1]
    dsem = ring[3 * NBUF + 1]
    ld = [None] * NBUF
    st = [None] * NBUF

    def start_load(j):
        b = j % NBUF
        ld[b] = pltpu.async_copy(
            x_hbm.at[pl.ds(base + half + j * RPD, RPD)], bufs[b], lsems[b])

    for j in range(NBUF):
        start_load(j)

    pltpu.sync_copy(idx_hbm.at[pl.ds(base, TPW1)], idx_v)

    def vreg_body(j, cnt):
        ev = idx_v[pl.ds(j * L, L)]
        rank = jnp.zeros((L,), jnp.int32)
        for k in range(L):
            ek = jnp.sum(jnp.where(lane == k, ev, 0))
            is_ek = (lane == ek)
            rk = jnp.sum(jnp.where(is_ek, cnt, 0))
            rank = jnp.where(lane == k, rk, rank)
            cnt = cnt + is_ek.astype(jnp.int32)
        rank_v[pl.ds(j * L, L)] = rank
        return cnt

    cnt = lax.fori_loop(0, TPW1 // L, vreg_body, jnp.zeros((L,), jnp.int32))

    cnt_v[...] = cnt
    pltpu.sync_copy(cnt_v, shared_cnt.at[pl.ds(sid * L, L)])
    plsc.subcore_barrier()
    pltpu.sync_copy(shared_cnt, allcnt_v)

    def acc_body(t, carry):
        tot, pri = carry
        row = allcnt_v[pl.ds(t * L, L)]
        tot = tot + row
        pri = pri + jnp.where(t < sid, row, 0)
        return tot, pri

    zeros = jnp.zeros((L,), jnp.int32)
    tot, pri = lax.fori_loop(0, NS, acc_body, (zeros, zeros))

    padded = ((tot + (BN - 1)) >> LOG_BN) << LOG_BN
    pend = plsc.cumsum(padded)
    pstart = pend - padded
    tb_v[...] = pstart + pri

    for j in range(TPW1 // L):
        ev = idx_v[pl.ds(j * L, L)]
        rk = rank_v[pl.ds(j * L, L)]
        dest_v[pl.ds(j * L, L)] = plsc.load_gather(tb_v, [ev]) + rk

    dest_wr = pltpu.async_copy(dest_v.at[pl.ds(half, TPW1 // 2)],
                               dest_hbm.at[pl.ds(base + half, TPW1 // 2)],
                               dsem)

    st_desc = [None] * nch
    waited = [False] * nch
    for j in range(nch):
        b = j % NBUF
        ld[b].wait()
        dv = dest_v[pl.ds(half + j * RPD, RPD)]
        st_desc[j] = pltpu.async_copy(bufs[b], x_pad_hbm.at[dv], ssems[b])
        nxt = j + NBUF - 1
        if NBUF <= nxt < nch:
            st_desc[j - 1].wait()
            waited[j - 1] = True
            start_load(nxt)
    for j in range(nch):
        if st_desc[j] is not None and not waited[j]:
            st_desc[j].wait()
    dest_wr.wait()

    @pl.when((sid == 0) & (cid == 0))
    def _():
        pend_blk = pend >> LOG_BN
        for v in range(NB // L):
            blk = lane + v * L
            acc = jnp.zeros((L,), jnp.int32)
            for e in range(E):
                pe = jnp.sum(jnp.where(lane == e, pend_blk, 0))
                acc = acc + (pe <= blk).astype(jnp.int32)
            bev[pl.ds(v * L, L)] = jnp.minimum(acc, E - 1)
        nblk = jnp.sum(jnp.where(lane == L - 1, pend_blk, 0))
        bev[pl.ds(NB, L)] = jnp.where(lane == 0, nblk, 0)
        pltpu.sync_copy(bev, be_hbm)


def _gmm_body(be_ref, x_ref, w_ref, o_ref):
    @pl.when(pl.program_id(0) < be_ref[NB])
    def _():
        o_ref[...] = jnp.dot(x_ref[...], w_ref[0],
                             preferred_element_type=jnp.float32)


def _gather_body(dest_hbm, out_pad_hbm, out_hbm, dest_v, *ring):
    wid = lax.axis_index("s") * NC + lax.axis_index("c")
    base = wid * TPW3
    pltpu.sync_copy(dest_hbm.at[pl.ds(base, TPW3)], dest_v)

    nch = TPW3 // RPD
    bufs = ring[:NBUF]
    gsems = ring[NBUF:2 * NBUF]
    wsems = ring[2 * NBUF:3 * NBUF]
    g = [None] * NBUF

    def start_gather(j):
        b = j % NBUF
        dv = dest_v[pl.ds(j * RPD, RPD)]
        g[b] = pltpu.async_copy(out_pad_hbm.at[dv], bufs[b], gsems[b])

    for j in range(min(NBUF, nch)):
        start_gather(j)

    w_desc = [None] * nch
    waited = [False] * nch
    for j in range(nch):
        b = j % NBUF
        g[b].wait()
        w_desc[j] = pltpu.async_copy(
            bufs[b], out_hbm.at[pl.ds(base + j * RPD, RPD)], wsems[b])
        nxt = j + NBUF - 1
        if NBUF <= nxt < nch:
            w_desc[j - 1].wait()
            waited[j - 1] = True
            start_gather(nxt)
    for j in range(nch):
        if w_desc[j] is not None and not waited[j]:
            w_desc[j].wait()


@jax.jit
def kernel(x, indices, weight):
    route = pl.kernel(
        _route_body,
        out_type=(
            jax.ShapeDtypeStruct((P, D_IN), jnp.float32),
            jax.ShapeDtypeStruct((N,), jnp.int32),
            jax.ShapeDtypeStruct((NB + L,), jnp.int32),
        ),
        mesh=plsc.VectorSubcoreMesh(
            core_axis_name="c", subcore_axis_name="s", num_cores=NC),
        compiler_params=pltpu.CompilerParams(needs_layout_passes=False),
        scratch_types=(
            pltpu.VMEM((TPW1,), jnp.int32),
            pltpu.VMEM((TPW1,), jnp.int32),
            pltpu.VMEM((L,), jnp.int32),
            pltpu.VMEM((NS * L,), jnp.int32),
            pltpu.VMEM((L,), jnp.int32),
            pltpu.VMEM((TPW1,), jnp.int32),
            pltpu.VMEM((NB + L,), jnp.int32),
        ) + (pltpu.VMEM((RPD, D_IN), jnp.float32),) * NBUF
          + (pltpu.VMEM_SHARED((NS * L,), jnp.int32),)
          + (pltpu.SemaphoreType.DMA,) * (2 * NBUF + 1),
    )
    x_pad, dest, block_expert = route(indices, x)

    out_pad = pl.pallas_call(
        _gmm_body,
        grid_spec=pltpu.PrefetchScalarGridSpec(
            num_scalar_prefetch=1,
            grid=(NB,),
            in_specs=[
                pl.BlockSpec((BN, D_IN),
                             lambda i, be: (jnp.minimum(i, be[NB] - 1), 0)),
                pl.BlockSpec((1, D_IN, D_OUT),
                             lambda i, be: (be[jnp.minimum(i, be[NB] - 1)],
                                            0, 0)),
            ],
            out_specs=pl.BlockSpec((BN, D_OUT),
                                   lambda i, be: (jnp.minimum(i, be[NB] - 1),
                                                  0)),
        ),
        out_shape=jax.ShapeDtypeStruct((P, D_OUT), jnp.float32),
    )(block_expert, x_pad, weight)

    gather_back = pl.kernel(
        _gather_body,
        out_type=jax.ShapeDtypeStruct((N, D_OUT), jnp.float32),
        mesh=plsc.VectorSubcoreMesh(
            core_axis_name="c", subcore_axis_name="s", num_cores=NC),
        compiler_params=pltpu.CompilerParams(needs_layout_passes=False),
        scratch_types=(
            pltpu.VMEM((TPW3,), jnp.int32),
        ) + (pltpu.VMEM((RPD, D_OUT), jnp.float32),) * NBUF
          + (pltpu.SemaphoreType.DMA,) * (2 * NBUF),
    )
    return gather_back(dest, out_pad)
